# baseline (speedup 1.0000x reference)
import functools

import jax
import jax.numpy as jnp
from jax import lax
from jax.experimental import pallas as pl
from jax.experimental.pallas import tpu as pltpu
from jax.experimental.pallas import tpu_sc as plsc

_B = 16384
_V = 100000
_D0, _D1, _D2 = 32, 64, 128
_AGG = 128
_S0, _S1 = 32768, 65536
_NC, _NS, _L = 2, 16, 16
_NW = _NC * _NS
_BPW = _B // _NW
_CH = 128
_NCH = _BPW // _CH
_NSET = 2

_LAST_BLK = (_V - 1) // 128


def _make_pack(d, pack, stride):
    def body(*refs):
        ins, out_ref = refs[:-1], refs[-1]
        parts = [jnp.transpose(r[...]) for r in ins]
        out_ref[...] = jnp.concatenate(parts, axis=1)

    def spec(j):
        return pl.BlockSpec(
            (d, 128),
            lambda i, _j=j: (0, jnp.minimum(_j * (stride // 128) + i, _LAST_BLK)),
        )

    return pl.pallas_call(
        body,
        grid=(stride // 128,),
        in_specs=[spec(j) for j in range(pack)],
        out_specs=pl.BlockSpec((128, d * pack), lambda i: (i, 0)),
        out_shape=jax.ShapeDtypeStruct((stride, d * pack), jnp.float32),
    )


def _sc_mean_reduction(indexes2d, q0, q1, t2):
    mesh = plsc.VectorSubcoreMesh(core_axis_name="c", subcore_axis_name="s")

    bufs = []
    for _ in range(_NSET):
        bufs.extend([
            pltpu.VMEM((_CH, _AGG), jnp.float32),
            pltpu.VMEM((_CH, _AGG), jnp.float32),
            pltpu.VMEM((_CH, _AGG), jnp.float32),
        ])

    @functools.partial(
        pl.kernel,
        mesh=mesh,
        out_type=jax.ShapeDtypeStruct((_B, _AGG), jnp.float32),
        compiler_params=pltpu.CompilerParams(use_tc_tiling_on_sc=False),
        scratch_types=[
            pltpu.VMEM((_NCH, _CH), jnp.int32),
            pltpu.VMEM((_NCH, _CH), jnp.int32),
            pltpu.VMEM((_NCH, _CH), jnp.int32),
        ]
        + bufs
        + [pltpu.SemaphoreType.DMA] * _NSET
        + [pltpu.SemaphoreType.DMA],
    )
    def run(idx_hbm, q0_hbm, q1_hbm, t2_hbm, out_hbm, idx_v, idx0_v, idx1_v,
            *scratch):
        gbufs = [scratch[s * 3:s * 3 + 3] for s in range(_NSET)]
        sems_in = scratch[_NSET * 3:_NSET * 3 + _NSET]
        sem_out = scratch[_NSET * 3 + _NSET]

        wid = lax.axis_index("s") * _NC + lax.axis_index("c")
        base = wid * _BPW

        pltpu.sync_copy(idx_hbm.at[pl.ds(wid * _NCH, _NCH)], idx_v)

        for c in range(_NCH):
            for jj in range(_CH // _L):
                cols = pl.ds(jj * _L, _L)
                iv = idx_v[c, cols]
                idx0_v[c, cols] = iv & (_S0 - 1)
                idx1_v[c, cols] = iv & (_S1 - 1)

        srcs = ((q0_hbm, idx0_v), (q1_hbm, idx1_v), (t2_hbm, idx_v))
        in_handles = [None] * _NCH
        out_handles = [None] * _NCH

        def fire_in(c):
            s = c % _NSET
            in_handles[c] = [
                pltpu.async_copy(tab.at[ivs.at[c]], gbufs[s][t], sems_in[s])
                for t, (tab, ivs) in enumerate(srcs)
            ]

        third = jnp.float32(1.0 / 3.0)

        def combine(c):
            s = c % _NSET
            g0, g1, g2 = gbufs[s]

            def body(g, carry):
                ivg = idx_v[c, pl.ds(g * _L, _L)]
                for l in range(_L):
                    ix = ivg[l]
                    o0 = lax.shift_right_logical(ix, 15) * _D0
                    o1 = lax.shift_right_logical(ix, 16) * _D1
                    r = g * _L + l
                    for j in range(_AGG // _L):
                        cols = pl.ds(j * _L, _L)
                        v = g2[r, cols]
                        if j * _L < _D0:
                            v = v + g0[r, pl.ds(o0 + j * _L, _L)]
                        if j * _L < _D1:
                            v = v + g1[r, pl.ds(o1 + j * _L, _L)]
                        g2[r, cols] = v * third
                return carry

            lax.fori_loop(0, _CH // _L, body, 0)

        fire_in(0)
        for c in range(_NCH):
            for h in in_handles[c]:
                h.wait()
            if c >= 1:
                out_handles[c - 1].wait()
            if c + 1 < _NCH:
                fire_in(c + 1)
            combine(c)
            out_handles[c] = pltpu.async_copy(
                gbufs[c % _NSET][2],
                out_hbm.at[pl.ds(base + c * _CH, _CH)],
                sem_out,
            )
        out_handles[_NCH - 1].wait()

    return run(indexes2d, q0, q1, t2)


def kernel(indexes, table0, table1, table2):
    idx2d = indexes.reshape(_NW * _NCH, _CH)
    q0 = _make_pack(_D0, _AGG // _D0, _S0)(*([table0.T] * (_AGG // _D0)))
    q1 = _make_pack(_D1, _AGG // _D1, _S1)(*([table1.T] * (_AGG // _D1)))
    return _sc_mean_reduction(idx2d, q0, q1, table2)


# speedup vs baseline: 2.6315x; 2.6315x over previous
import functools

import jax
import jax.numpy as jnp
from jax import lax
from jax.experimental import pallas as pl
from jax.experimental.pallas import tpu as pltpu
from jax.experimental.pallas import tpu_sc as plsc

_B = 16384
_V = 100000
_D0, _D1, _D2 = 32, 64, 128
_AGG = 128
_S0, _S1 = 32768, 65536
_NC, _NS, _L = 2, 16, 16
_NW = _NC * _NS
_BPW = _B // _NW
_CH = 128
_NCH = _BPW // _CH
_NSET = 2

_PBLK = 2048
_LAST_PBLK = (_V - 1) // _PBLK


def _make_pack(d, pack, stride):
    def body(*refs):
        ins, out_ref = refs[:-1], refs[-1]
        eye = jnp.eye(d, dtype=jnp.float32)
        parts = [
            lax.dot_general(
                r[...], eye, (((0,), (0,)), ((), ())),
                preferred_element_type=jnp.float32,
                precision=lax.Precision.HIGHEST,
            )
            for r in ins
        ]
        out_ref[...] = jnp.concatenate(parts, axis=1)

    def spec(j):
        return pl.BlockSpec(
            (d, _PBLK),
            lambda i, _j=j: (0, jnp.minimum(_j * (stride // _PBLK) + i, _LAST_PBLK)),
        )

    return pl.pallas_call(
        body,
        grid=(stride // _PBLK,),
        in_specs=[spec(j) for j in range(pack)],
        out_specs=pl.BlockSpec((_PBLK, d * pack), lambda i: (i, 0)),
        out_shape=jax.ShapeDtypeStruct((stride, d * pack), jnp.float32),
    )


def _sc_mean_reduction(indexes2d, q0, q1, t2):
    mesh = plsc.VectorSubcoreMesh(core_axis_name="c", subcore_axis_name="s")

    bufs = []
    for _ in range(_NSET):
        bufs.extend([
            pltpu.VMEM((_CH, _AGG), jnp.float32),
            pltpu.VMEM((_CH, _AGG), jnp.float32),
            pltpu.VMEM((_CH, _AGG), jnp.float32),
        ])

    @functools.partial(
        pl.kernel,
        mesh=mesh,
        out_type=jax.ShapeDtypeStruct((_B, _AGG), jnp.float32),
        compiler_params=pltpu.CompilerParams(use_tc_tiling_on_sc=False),
        scratch_types=[
            pltpu.VMEM((_NCH, _CH), jnp.int32),
            pltpu.VMEM((_NCH, _CH), jnp.int32),
            pltpu.VMEM((_NCH, _CH), jnp.int32),
        ]
        + bufs
        + [pltpu.SemaphoreType.DMA] * _NSET
        + [pltpu.SemaphoreType.DMA],
    )
    def run(idx_hbm, q0_hbm, q1_hbm, t2_hbm, out_hbm, idx_v, idx0_v, idx1_v,
            *scratch):
        gbufs = [scratch[s * 3:s * 3 + 3] for s in range(_NSET)]
        sems_in = scratch[_NSET * 3:_NSET * 3 + _NSET]
        sem_out = scratch[_NSET * 3 + _NSET]

        wid = lax.axis_index("s") * _NC + lax.axis_index("c")
        base = wid * _BPW

        pltpu.sync_copy(idx_hbm.at[pl.ds(wid * _NCH, _NCH)], idx_v)

        for c in range(_NCH):
            for jj in range(_CH // _L):
                cols = pl.ds(jj * _L, _L)
                iv = idx_v[c, cols]
                idx0_v[c, cols] = iv & (_S0 - 1)
                idx1_v[c, cols] = iv & (_S1 - 1)

        srcs = ((q0_hbm, idx0_v), (q1_hbm, idx1_v), (t2_hbm, idx_v))
        in_handles = [None] * _NCH
        out_handles = [None] * _NCH

        def fire_in(c):
            s = c % _NSET
            in_handles[c] = [
                pltpu.async_copy(tab.at[ivs.at[c]], gbufs[s][t], sems_in[s])
                for t, (tab, ivs) in enumerate(srcs)
            ]

        third = jnp.float32(1.0 / 3.0)

        def combine(c):
            s = c % _NSET
            g0, g1, g2 = gbufs[s]

            def body(g, carry):
                ivg = idx_v[c, pl.ds(g * _L, _L)]
                for l in range(_L):
                    ix = ivg[l]
                    o0 = lax.shift_right_logical(ix, 15) * _D0
                    o1 = lax.shift_right_logical(ix, 16) * _D1
                    r = g * _L + l
                    for j in range(_AGG // _L):
                        cols = pl.ds(j * _L, _L)
                        v = g2[r, cols]
                        if j * _L < _D0:
                            v = v + g0[r, pl.ds(o0 + j * _L, _L)]
                        if j * _L < _D1:
                            v = v + g1[r, pl.ds(o1 + j * _L, _L)]
                        g2[r, cols] = v * third
                return carry

            lax.fori_loop(0, _CH // _L, body, 0)

        fire_in(0)
        for c in range(_NCH):
            for h in in_handles[c]:
                h.wait()
            if c >= 1:
                out_handles[c - 1].wait()
            if c + 1 < _NCH:
                fire_in(c + 1)
            combine(c)
            out_handles[c] = pltpu.async_copy(
                gbufs[c % _NSET][2],
                out_hbm.at[pl.ds(base + c * _CH, _CH)],
                sem_out,
            )
        out_handles[_NCH - 1].wait()

    return run(indexes2d, q0, q1, t2)


def kernel(indexes, table0, table1, table2):
    idx2d = indexes.reshape(_NW * _NCH, _CH)
    q0 = _make_pack(_D0, _AGG // _D0, _S0)(*([table0.T] * (_AGG // _D0)))
    q1 = _make_pack(_D1, _AGG // _D1, _S1)(*([table1.T] * (_AGG // _D1)))
    return _sc_mean_reduction(idx2d, q0, q1, table2)


# speedup vs baseline: 5.1900x; 1.9723x over previous
import functools

import jax
import jax.numpy as jnp
from jax import lax
from jax.experimental import pallas as pl
from jax.experimental.pallas import tpu as pltpu
from jax.experimental.pallas import tpu_sc as plsc

_B = 16384
_V = 100000
_D0, _D1, _D2 = 32, 64, 128
_AGG = 128
_S0, _S1 = 32768, 65536
_NC, _NS, _L = 2, 16, 16
_NW = _NC * _NS
_BPW = _B // _NW
_CH = 128
_NCH = _BPW // _CH
_NSET = 2

_PBLK = 2048
_LAST_PBLK = (_V - 1) // _PBLK


def _make_pack(d, pack, stride):
    def body(*refs):
        ins, out_ref = refs[:-1], refs[-1]
        stacked = jnp.concatenate([r[...] for r in ins], axis=0)
        out_ref[...] = jnp.transpose(stacked)

    def spec(j):
        return pl.BlockSpec(
            (d, _PBLK),
            lambda i, _j=j: (0, jnp.minimum(_j * (stride // _PBLK) + i, _LAST_PBLK)),
        )

    return pl.pallas_call(
        body,
        grid=(stride // _PBLK,),
        in_specs=[spec(j) for j in range(pack)],
        out_specs=pl.BlockSpec((_PBLK, d * pack), lambda i: (i, 0)),
        out_shape=jax.ShapeDtypeStruct((stride, d * pack), jnp.float32),
    )


def _sc_mean_reduction(indexes2d, q0, q1, t2):
    mesh = plsc.VectorSubcoreMesh(core_axis_name="c", subcore_axis_name="s")

    bufs = []
    for _ in range(_NSET):
        bufs.extend([
            pltpu.VMEM((_CH, _AGG), jnp.float32),
            pltpu.VMEM((_CH, _AGG), jnp.float32),
            pltpu.VMEM((_CH, _AGG), jnp.float32),
        ])

    @functools.partial(
        pl.kernel,
        mesh=mesh,
        out_type=jax.ShapeDtypeStruct((_B, _AGG), jnp.float32),
        compiler_params=pltpu.CompilerParams(use_tc_tiling_on_sc=False),
        scratch_types=[
            pltpu.VMEM((_NCH, _CH), jnp.int32),
            pltpu.VMEM((_NCH, _CH), jnp.int32),
            pltpu.VMEM((_NCH, _CH), jnp.int32),
        ]
        + bufs
        + [pltpu.SemaphoreType.DMA] * _NSET
        + [pltpu.SemaphoreType.DMA],
    )
    def run(idx_hbm, q0_hbm, q1_hbm, t2_hbm, out_hbm, idx_v, idx0_v, idx1_v,
            *scratch):
        gbufs = [scratch[s * 3:s * 3 + 3] for s in range(_NSET)]
        sems_in = scratch[_NSET * 3:_NSET * 3 + _NSET]
        sem_out = scratch[_NSET * 3 + _NSET]

        wid = lax.axis_index("s") * _NC + lax.axis_index("c")
        base = wid * _BPW

        pltpu.sync_copy(idx_hbm.at[pl.ds(wid * _NCH, _NCH)], idx_v)

        for c in range(_NCH):
            for jj in range(_CH // _L):
                cols = pl.ds(jj * _L, _L)
                iv = idx_v[c, cols]
                idx0_v[c, cols] = iv & (_S0 - 1)
                idx1_v[c, cols] = iv & (_S1 - 1)

        srcs = ((q0_hbm, idx0_v), (q1_hbm, idx1_v), (t2_hbm, idx_v))
        in_handles = [None] * _NCH
        out_handles = [None] * _NCH

        def fire_in(c):
            s = c % _NSET
            in_handles[c] = [
                pltpu.async_copy(tab.at[ivs.at[c]], gbufs[s][t], sems_in[s])
                for t, (tab, ivs) in enumerate(srcs)
            ]

        third = jnp.float32(1.0 / 3.0)

        def combine(c):
            s = c % _NSET
            g0, g1, g2 = gbufs[s]

            def body(g, carry):
                ivg = idx_v[c, pl.ds(g * _L, _L)]
                for l in range(_L):
                    ix = ivg[l]
                    o0 = lax.shift_right_logical(ix, 15) * _D0
                    o1 = lax.shift_right_logical(ix, 16) * _D1
                    r = g * _L + l
                    for j in range(_AGG // _L):
                        cols = pl.ds(j * _L, _L)
                        v = g2[r, cols]
                        if j * _L < _D0:
                            v = v + g0[r, pl.ds(o0 + j * _L, _L)]
                        if j * _L < _D1:
                            v = v + g1[r, pl.ds(o1 + j * _L, _L)]
                        g2[r, cols] = v * third
                return carry

            lax.fori_loop(0, _CH // _L, body, 0)

        fire_in(0)
        for c in range(_NCH):
            for h in in_handles[c]:
                h.wait()
            if c >= 1:
                out_handles[c - 1].wait()
            if c + 1 < _NCH:
                fire_in(c + 1)
            combine(c)
            out_handles[c] = pltpu.async_copy(
                gbufs[c % _NSET][2],
                out_hbm.at[pl.ds(base + c * _CH, _CH)],
                sem_out,
            )
        out_handles[_NCH - 1].wait()

    return run(indexes2d, q0, q1, t2)


def kernel(indexes, table0, table1, table2):
    idx2d = indexes.reshape(_NW * _NCH, _CH)
    q0 = _make_pack(_D0, _AGG // _D0, _S0)(*([table0.T] * (_AGG // _D0)))
    q1 = _make_pack(_D1, _AGG // _D1, _S1)(*([table1.T] * (_AGG // _D1)))
    return _sc_mean_reduction(idx2d, q0, q1, table2)


# speedup vs baseline: 5.3539x; 1.0316x over previous
import functools

import jax
import jax.numpy as jnp
from jax import lax
from jax.experimental import pallas as pl
from jax.experimental.pallas import tpu as pltpu
from jax.experimental.pallas import tpu_sc as plsc

_B = 16384
_V = 100000
_D0, _D1, _D2 = 32, 64, 128
_AGG = 128
_S0, _S1 = 32768, 65536
_NC, _NS, _L = 2, 16, 16
_NW = _NC * _NS
_BPW = _B // _NW
_CH = 128
_NCH = _BPW // _CH
_NSET = 2

_PBLK = 2048
_LAST_PBLK = (_V - 1) // _PBLK


def _make_pack(d, pack, stride):
    def body(*refs):
        ins, out_ref = refs[:-1], refs[-1]
        stacked = jnp.concatenate([r[...] for r in ins], axis=0)
        out_ref[...] = jnp.transpose(stacked)

    def spec(j):
        return pl.BlockSpec(
            (d, _PBLK),
            lambda i, _j=j: (0, jnp.minimum(_j * (stride // _PBLK) + i, _LAST_PBLK)),
        )

    return pl.pallas_call(
        body,
        grid=(stride // _PBLK,),
        in_specs=[spec(j) for j in range(pack)],
        out_specs=pl.BlockSpec((_PBLK, d * pack), lambda i: (i, 0)),
        out_shape=jax.ShapeDtypeStruct((stride, d * pack), jnp.float32),
    )


def _sc_mean_reduction(indexes2d, q0, q1, t2):
    mesh = plsc.VectorSubcoreMesh(core_axis_name="c", subcore_axis_name="s")

    bufs = []
    for _ in range(_NSET):
        bufs.extend([
            pltpu.VMEM((_CH, _D0), jnp.float32),
            pltpu.VMEM((_CH, _D1), jnp.float32),
            pltpu.VMEM((_CH, _D2), jnp.float32),
        ])

    @functools.partial(
        pl.kernel,
        mesh=mesh,
        out_type=jax.ShapeDtypeStruct((_B, _AGG), jnp.float32),
        compiler_params=pltpu.CompilerParams(use_tc_tiling_on_sc=False),
        scratch_types=[
            pltpu.VMEM((_NCH, _CH), jnp.int32),
            pltpu.VMEM((_NCH, _CH), jnp.int32),
            pltpu.VMEM((_NCH, _CH), jnp.int32),
        ]
        + bufs
        + [pltpu.SemaphoreType.DMA] * _NSET
        + [pltpu.SemaphoreType.DMA],
    )
    def run(idx_hbm, q0_hbm, q1_hbm, t2_hbm, out_hbm, idx_v, idx0_v, idx1_v,
            *scratch):
        gbufs = [scratch[s * 3:s * 3 + 3] for s in range(_NSET)]
        sems_in = scratch[_NSET * 3:_NSET * 3 + _NSET]
        sem_out = scratch[_NSET * 3 + _NSET]

        wid = lax.axis_index("s") * _NC + lax.axis_index("c")
        base = wid * _BPW

        pltpu.sync_copy(idx_hbm.at[pl.ds(wid * _NCH, _NCH)], idx_v)

        for c in range(_NCH):
            for jj in range(_CH // _L):
                cols = pl.ds(jj * _L, _L)
                iv = idx_v[c, cols]
                idx0_v[c, cols] = lax.shift_left(iv & (_S0 - 1), 2) | (
                    lax.shift_right_logical(iv, 15))
                idx1_v[c, cols] = lax.shift_left(iv & (_S1 - 1), 1) | (
                    lax.shift_right_logical(iv, 16))

        srcs = ((q0_hbm, idx0_v), (q1_hbm, idx1_v), (t2_hbm, idx_v))
        in_handles = [None] * _NCH
        out_handles = [None] * _NCH

        def fire_in(c):
            s = c % _NSET
            in_handles[c] = [
                pltpu.async_copy(tab.at[ivs.at[c]], gbufs[s][t], sems_in[s])
                for t, (tab, ivs) in enumerate(srcs)
            ]

        third = jnp.float32(1.0 / 3.0)

        def combine(c):
            s = c % _NSET
            g0, g1, g2 = gbufs[s]

            def body(r, carry):
                for j in range(_AGG // _L):
                    cols = pl.ds(j * _L, _L)
                    v = g2[r, cols]
                    if j * _L < _D0:
                        v = v + g0[r, cols]
                    if j * _L < _D1:
                        v = v + g1[r, cols]
                    g2[r, cols] = v * third
                return carry

            lax.fori_loop(0, _CH, body, 0)

        fire_in(0)
        for c in range(_NCH):
            for h in in_handles[c]:
                h.wait()
            if c >= 1:
                out_handles[c - 1].wait()
            if c + 1 < _NCH:
                fire_in(c + 1)
            combine(c)
            out_handles[c] = pltpu.async_copy(
                gbufs[c % _NSET][2],
                out_hbm.at[pl.ds(base + c * _CH, _CH)],
                sem_out,
            )
        out_handles[_NCH - 1].wait()

    return run(indexes2d, q0, q1, t2)


def kernel(indexes, table0, table1, table2):
    idx2d = indexes.reshape(_NW * _NCH, _CH)
    q0 = _make_pack(_D0, _AGG // _D0, _S0)(*([table0.T] * (_AGG // _D0)))
    q1 = _make_pack(_D1, _AGG // _D1, _S1)(*([table1.T] * (_AGG // _D1)))
    v0 = q0.reshape(_S0 * (_AGG // _D0), _D0)
    v1 = q1.reshape(_S1 * (_AGG // _D1), _D1)
    return _sc_mean_reduction(idx2d, v0, v1, table2)


# speedup vs baseline: 5.9530x; 1.1119x over previous
import functools

import jax
import jax.numpy as jnp
from jax import lax
from jax.experimental import pallas as pl
from jax.experimental.pallas import tpu as pltpu
from jax.experimental.pallas import tpu_sc as plsc

_B = 16384
_V = 100000
_D0, _D1, _D2 = 32, 64, 128
_AGG = 128
_S0, _S1 = 26624, 51200
_NC, _NS, _L = 2, 16, 16
_NW = _NC * _NS
_BPW = _B // _NW
_CH = 128
_NCH = _BPW // _CH
_NSET = 2

_PBLK = 2048
_LAST_PBLK = (_V - 1) // _PBLK


def _make_pack(d, pack, stride):
    def body(*refs):
        ins, out_ref = refs[:-1], refs[-1]
        stacked = jnp.concatenate([r[...] for r in ins], axis=0)
        out_ref[...] = jnp.transpose(stacked)

    def spec(j):
        return pl.BlockSpec(
            (d, _PBLK),
            lambda i, _j=j: (0, jnp.minimum(_j * (stride // _PBLK) + i, _LAST_PBLK)),
        )

    return pl.pallas_call(
        body,
        grid=(stride // _PBLK,),
        in_specs=[spec(j) for j in range(pack)],
        out_specs=pl.BlockSpec((_PBLK, d * pack), lambda i: (i, 0)),
        out_shape=jax.ShapeDtypeStruct((stride, d * pack), jnp.float32),
    )


def _sc_mean_reduction(indexes2d, q0, q1, t2):
    mesh = plsc.VectorSubcoreMesh(core_axis_name="c", subcore_axis_name="s")

    bufs = []
    for _ in range(_NSET):
        bufs.extend([
            pltpu.VMEM((_CH, _D0), jnp.float32),
            pltpu.VMEM((_CH, _D1), jnp.float32),
            pltpu.VMEM((_CH, _D2), jnp.float32),
        ])

    @functools.partial(
        pl.kernel,
        mesh=mesh,
        out_type=jax.ShapeDtypeStruct((_B, _AGG), jnp.float32),
        compiler_params=pltpu.CompilerParams(use_tc_tiling_on_sc=False),
        scratch_types=[
            pltpu.VMEM((_NCH, _CH), jnp.int32),
            pltpu.VMEM((_NCH, _CH), jnp.int32),
            pltpu.VMEM((_NCH, _CH), jnp.int32),
        ]
        + bufs
        + [pltpu.SemaphoreType.DMA] * _NSET
        + [pltpu.SemaphoreType.DMA],
    )
    def run(idx_hbm, q0_hbm, q1_hbm, t2_hbm, out_hbm, idx_v, idx0_v, idx1_v,
            *scratch):
        gbufs = [scratch[s * 3:s * 3 + 3] for s in range(_NSET)]
        sems_in = scratch[_NSET * 3:_NSET * 3 + _NSET]
        sem_out = scratch[_NSET * 3 + _NSET]

        wid = lax.axis_index("s") * _NC + lax.axis_index("c")
        base = wid * _BPW

        pltpu.sync_copy(idx_hbm.at[pl.ds(wid * _NCH, _NCH)], idx_v)

        ones = jnp.full((_L,), 1, jnp.int32)
        zeros = jnp.full((_L,), 0, jnp.int32)
        for c in range(_NCH):
            for jj in range(_CH // _L):
                cols = pl.ds(jj * _L, _L)
                iv = idx_v[c, cols]
                j0 = (jnp.where(iv >= _S0, ones, zeros)
                      + jnp.where(iv >= 2 * _S0, ones, zeros)
                      + jnp.where(iv >= 3 * _S0, ones, zeros))
                j1 = jnp.where(iv >= _S1, ones, zeros)
                idx0_v[c, cols] = (iv - j0 * _S0) * 4 + j0
                idx1_v[c, cols] = (iv - j1 * _S1) * 2 + j1

        srcs = ((q0_hbm, idx0_v), (q1_hbm, idx1_v), (t2_hbm, idx_v))
        in_handles = [None] * _NCH
        out_handles = [None] * _NCH

        def fire_in(c):
            s = c % _NSET
            in_handles[c] = [
                pltpu.async_copy(tab.at[ivs.at[c]], gbufs[s][t], sems_in[s])
                for t, (tab, ivs) in enumerate(srcs)
            ]

        third = jnp.float32(1.0 / 3.0)

        def combine(c):
            s = c % _NSET
            g0, g1, g2 = gbufs[s]

            def body(r, carry):
                for j in range(_AGG // _L):
                    cols = pl.ds(j * _L, _L)
                    v = g2[r, cols]
                    if j * _L < _D0:
                        v = v + g0[r, cols]
                    if j * _L < _D1:
                        v = v + g1[r, cols]
                    g2[r, cols] = v * third
                return carry

            lax.fori_loop(0, _CH, body, 0)

        fire_in(0)
        for c in range(_NCH):
            for h in in_handles[c]:
                h.wait()
            if c >= 1:
                out_handles[c - 1].wait()
            if c + 1 < _NCH:
                fire_in(c + 1)
            combine(c)
            out_handles[c] = pltpu.async_copy(
                gbufs[c % _NSET][2],
                out_hbm.at[pl.ds(base + c * _CH, _CH)],
                sem_out,
            )
        out_handles[_NCH - 1].wait()

    return run(indexes2d, q0, q1, t2)


def kernel(indexes, table0, table1, table2):
    idx2d = indexes.reshape(_NW * _NCH, _CH)
    q0 = _make_pack(_D0, _AGG // _D0, _S0)(*([table0.T] * (_AGG // _D0)))
    q1 = _make_pack(_D1, _AGG // _D1, _S1)(*([table1.T] * (_AGG // _D1)))
    v0 = q0.reshape(_S0 * (_AGG // _D0), _D0)
    v1 = q1.reshape(_S1 * (_AGG // _D1), _D1)
    return _sc_mean_reduction(idx2d, v0, v1, table2)


# speedup vs baseline: 6.7537x; 1.1345x over previous
import functools

import jax
import jax.numpy as jnp
from jax import lax
from jax.experimental import pallas as pl
from jax.experimental.pallas import tpu as pltpu
from jax.experimental.pallas import tpu_sc as plsc

_B = 16384
_V = 100000
_D0, _D1, _D2 = 32, 64, 128
_AGG = 128
_S0, _S1 = 28672, 53248
_NC, _NS, _L = 2, 16, 16
_NW = _NC * _NS
_BPW = _B // _NW
_CH = 128
_NCH = _BPW // _CH
_NSET = 2

_PBLK = 4096
_LAST_PBLK = (_V - 1) // _PBLK


def _make_pack(d, pack, stride):
    def body(*refs):
        ins, out_ref = refs[:-1], refs[-1]
        stacked = jnp.concatenate([r[...] for r in ins], axis=0)
        out_ref[...] = jnp.transpose(stacked)

    def spec(j):
        return pl.BlockSpec(
            (d, _PBLK),
            lambda i, _j=j: (0, jnp.minimum(_j * (stride // _PBLK) + i, _LAST_PBLK)),
        )

    return pl.pallas_call(
        body,
        grid=(stride // _PBLK,),
        in_specs=[spec(j) for j in range(pack)],
        out_specs=pl.BlockSpec((_PBLK, d * pack), lambda i: (i, 0)),
        out_shape=jax.ShapeDtypeStruct((stride, d * pack), jnp.float32),
    )


def _sc_mean_reduction(indexes2d, q0, q1, t2):
    mesh = plsc.VectorSubcoreMesh(core_axis_name="c", subcore_axis_name="s")

    bufs = []
    for _ in range(_NSET):
        bufs.extend([
            pltpu.VMEM((_CH, _D0), jnp.float32),
            pltpu.VMEM((_CH, _D1), jnp.float32),
            pltpu.VMEM((_CH, _D2), jnp.float32),
        ])

    @functools.partial(
        pl.kernel,
        mesh=mesh,
        out_type=jax.ShapeDtypeStruct((_B, _AGG), jnp.float32),
        compiler_params=pltpu.CompilerParams(use_tc_tiling_on_sc=False),
        scratch_types=[
            pltpu.VMEM((_NCH, _CH), jnp.int32),
            pltpu.VMEM((_NCH, _CH), jnp.int32),
            pltpu.VMEM((_NCH, _CH), jnp.int32),
        ]
        + bufs
        + [pltpu.SemaphoreType.DMA] * _NSET
        + [pltpu.SemaphoreType.DMA],
    )
    def run(idx_hbm, q0_hbm, q1_hbm, t2_hbm, out_hbm, idx_v, idx0_v, idx1_v,
            *scratch):
        gbufs = [scratch[s * 3:s * 3 + 3] for s in range(_NSET)]
        sems_in = scratch[_NSET * 3:_NSET * 3 + _NSET]
        sem_out = scratch[_NSET * 3 + _NSET]

        wid = lax.axis_index("s") * _NC + lax.axis_index("c")
        base = wid * _BPW

        pltpu.sync_copy(idx_hbm.at[pl.ds(wid * _NCH, _NCH)], idx_v)

        ones = jnp.full((_L,), 1, jnp.int32)
        zeros = jnp.full((_L,), 0, jnp.int32)
        for c in range(_NCH):
            for jj in range(_CH // _L):
                cols = pl.ds(jj * _L, _L)
                iv = idx_v[c, cols]
                j0 = (jnp.where(iv >= _S0, ones, zeros)
                      + jnp.where(iv >= 2 * _S0, ones, zeros)
                      + jnp.where(iv >= 3 * _S0, ones, zeros))
                j1 = jnp.where(iv >= _S1, ones, zeros)
                idx0_v[c, cols] = (iv - j0 * _S0) * 4 + j0
                idx1_v[c, cols] = (iv - j1 * _S1) * 2 + j1

        srcs = ((q0_hbm, idx0_v), (q1_hbm, idx1_v), (t2_hbm, idx_v))
        in_handles = [None] * _NCH
        out_handles = [None] * _NCH

        def fire_in(c):
            s = c % _NSET
            in_handles[c] = [
                pltpu.async_copy(tab.at[ivs.at[c]], gbufs[s][t], sems_in[s])
                for t, (tab, ivs) in enumerate(srcs)
            ]

        third = jnp.float32(1.0 / 3.0)

        def combine(c):
            s = c % _NSET
            g0, g1, g2 = gbufs[s]

            def body(h, carry):
                for u in range(2):
                    r = h * 2 + u
                    for j in range(_AGG // _L):
                        cols = pl.ds(j * _L, _L)
                        v = g2[r, cols]
                        if j * _L < _D0:
                            v = v + g0[r, cols]
                        if j * _L < _D1:
                            v = v + g1[r, cols]
                        g2[r, cols] = v * third
                return carry

            lax.fori_loop(0, _CH // 2, body, 0)

        fire_in(0)
        for c in range(_NCH):
            for h in in_handles[c]:
                h.wait()
            if c >= 1:
                out_handles[c - 1].wait()
            if c + 1 < _NCH:
                fire_in(c + 1)
            combine(c)
            out_handles[c] = pltpu.async_copy(
                gbufs[c % _NSET][2],
                out_hbm.at[pl.ds(base + c * _CH, _CH)],
                sem_out,
            )
        out_handles[_NCH - 1].wait()

    return run(indexes2d, q0, q1, t2)


def kernel(indexes, table0, table1, table2):
    idx2d = indexes.reshape(_NW * _NCH, _CH)
    q0 = _make_pack(_D0, _AGG // _D0, _S0)(*([table0.T] * (_AGG // _D0)))
    q1 = _make_pack(_D1, _AGG // _D1, _S1)(*([table1.T] * (_AGG // _D1)))
    v0 = q0.reshape(_S0 * (_AGG // _D0), _D0)
    v1 = q1.reshape(_S1 * (_AGG // _D1), _D1)
    return _sc_mean_reduction(idx2d, v0, v1, table2)


# speedup vs baseline: 6.9599x; 1.0305x over previous
import functools

import jax
import jax.numpy as jnp
from jax import lax
from jax.experimental import pallas as pl
from jax.experimental.pallas import tpu as pltpu
from jax.experimental.pallas import tpu_sc as plsc

_B = 16384
_V = 100000
_D0, _D1, _D2 = 32, 64, 128
_AGG = 128
_S0, _S1 = 32768, 57344
_NC, _NS, _L = 2, 16, 16
_NW = _NC * _NS
_BPW = _B // _NW
_CH = 128
_NCH = _BPW // _CH
_NSET = 2

_PBLK = 8192
_LAST_PBLK = (_V - 1) // _PBLK


def _make_pack(d, pack, stride):
    def body(*refs):
        ins, out_ref = refs[:-1], refs[-1]
        stacked = jnp.concatenate([r[...] for r in ins], axis=0)
        out_ref[...] = jnp.transpose(stacked)

    def spec(j):
        return pl.BlockSpec(
            (d, _PBLK),
            lambda i, _j=j: (0, jnp.minimum(_j * (stride // _PBLK) + i, _LAST_PBLK)),
        )

    return pl.pallas_call(
        body,
        grid=(stride // _PBLK,),
        in_specs=[spec(j) for j in range(pack)],
        out_specs=pl.BlockSpec((_PBLK, d * pack), lambda i: (i, 0)),
        out_shape=jax.ShapeDtypeStruct((stride, d * pack), jnp.float32),
    )


def _sc_mean_reduction(indexes2d, q0, q1, t2):
    mesh = plsc.VectorSubcoreMesh(core_axis_name="c", subcore_axis_name="s")

    bufs = []
    for _ in range(_NSET):
        bufs.extend([
            pltpu.VMEM((_CH, _D0), jnp.float32),
            pltpu.VMEM((_CH, _D1), jnp.float32),
            pltpu.VMEM((_CH, _D2), jnp.float32),
        ])

    @functools.partial(
        pl.kernel,
        mesh=mesh,
        out_type=jax.ShapeDtypeStruct((_B, _AGG), jnp.float32),
        compiler_params=pltpu.CompilerParams(use_tc_tiling_on_sc=False),
        scratch_types=[
            pltpu.VMEM((_NCH, _CH), jnp.int32),
            pltpu.VMEM((_NCH, _CH), jnp.int32),
            pltpu.VMEM((_NCH, _CH), jnp.int32),
        ]
        + bufs
        + [pltpu.SemaphoreType.DMA] * _NSET
        + [pltpu.SemaphoreType.DMA],
    )
    def run(idx_hbm, q0_hbm, q1_hbm, t2_hbm, out_hbm, idx_v, idx0_v, idx1_v,
            *scratch):
        gbufs = [scratch[s * 3:s * 3 + 3] for s in range(_NSET)]
        sems_in = scratch[_NSET * 3:_NSET * 3 + _NSET]
        sem_out = scratch[_NSET * 3 + _NSET]

        wid = lax.axis_index("s") * _NC + lax.axis_index("c")
        base = wid * _BPW

        pltpu.sync_copy(idx_hbm.at[pl.ds(wid * _NCH, _NCH)], idx_v)

        ones = jnp.full((_L,), 1, jnp.int32)
        zeros = jnp.full((_L,), 0, jnp.int32)
        for c in range(_NCH):
            for jj in range(_CH // _L):
                cols = pl.ds(jj * _L, _L)
                iv = idx_v[c, cols]
                j0 = (jnp.where(iv >= _S0, ones, zeros)
                      + jnp.where(iv >= 2 * _S0, ones, zeros)
                      + jnp.where(iv >= 3 * _S0, ones, zeros))
                j1 = jnp.where(iv >= _S1, ones, zeros)
                idx0_v[c, cols] = (iv - j0 * _S0) * 4 + j0
                idx1_v[c, cols] = (iv - j1 * _S1) * 2 + j1

        srcs = ((q0_hbm, idx0_v), (q1_hbm, idx1_v), (t2_hbm, idx_v))
        in_handles = [None] * _NCH
        out_handles = [None] * _NCH

        def fire_in(c):
            s = c % _NSET
            in_handles[c] = [
                pltpu.async_copy(tab.at[ivs.at[c]], gbufs[s][t], sems_in[s])
                for t, (tab, ivs) in enumerate(srcs)
            ]

        third = jnp.float32(1.0 / 3.0)

        def combine(c):
            s = c % _NSET
            g0, g1, g2 = gbufs[s]

            def body(h, carry):
                for u in range(4):
                    r = h * 4 + u
                    for j in range(_AGG // _L):
                        cols = pl.ds(j * _L, _L)
                        v = g2[r, cols]
                        if j * _L < _D0:
                            v = v + g0[r, cols]
                        if j * _L < _D1:
                            v = v + g1[r, cols]
                        g2[r, cols] = v * third
                return carry

            lax.fori_loop(0, _CH // 4, body, 0)

        fire_in(0)
        for c in range(_NCH):
            for h in in_handles[c]:
                h.wait()
            if c >= 1:
                out_handles[c - 1].wait()
            if c + 1 < _NCH:
                fire_in(c + 1)
            combine(c)
            out_handles[c] = pltpu.async_copy(
                gbufs[c % _NSET][2],
                out_hbm.at[pl.ds(base + c * _CH, _CH)],
                sem_out,
            )
        out_handles[_NCH - 1].wait()

    return run(indexes2d, q0, q1, t2)


def kernel(indexes, table0, table1, table2):
    idx2d = indexes.reshape(_NW * _NCH, _CH)
    q0 = _make_pack(_D0, _AGG // _D0, _S0)(*([table0.T] * (_AGG // _D0)))
    q1 = _make_pack(_D1, _AGG // _D1, _S1)(*([table1.T] * (_AGG // _D1)))
    v0 = q0.reshape(_S0 * (_AGG // _D0), _D0)
    v1 = q1.reshape(_S1 * (_AGG // _D1), _D1)
    return _sc_mean_reduction(idx2d, v0, v1, table2)
